# baseline (speedup 1.0000x reference)
import functools

import jax
import jax.numpy as jnp
import numpy as np
from jax import lax
from jax.experimental import pallas as pl
from jax.experimental.pallas import tpu as pltpu
from jax.experimental.pallas import tpu_sc as plsc

NC = 2
NS = 16
NW = NC * NS
LANES = 16
K = 80


def _mesh():
  return plsc.VectorSubcoreMesh(core_axis_name="c", subcore_axis_name="s",
                                num_cores=NC)


@functools.partial(jax.jit, static_argnames=("n", "r", "pt"))
def _sc_prep(src, dst, typ, zeros_hist, *, n, r, pt):
  e = src.shape[0]
  nr = n * r
  per_tile_a = e // NS
  per_tile_b = e // NW
  nb_a = per_tile_a // K
  nb_b = per_tile_b // K
  stripe = nr // NS
  lanes_b = per_tile_b // LANES
  padw = pt - per_tile_b

  @functools.partial(
      pl.kernel,
      out_type=(jax.ShapeDtypeStruct((NW * pt,), jnp.int32),
                jax.ShapeDtypeStruct((NW * pt,), jnp.int32),
                jax.ShapeDtypeStruct((NW * pt,), jnp.float32)),
      mesh=_mesh(),
      scratch_types=[
          pltpu.VMEM((4 * per_tile_b,), jnp.int32),
          pltpu.VMEM((nb_a, K), jnp.int32),
          pltpu.VMEM((per_tile_b,), jnp.float32),
          pltpu.VMEM((per_tile_b,), jnp.float32),
          pltpu.VMEM((K,), jnp.float32),
          pltpu.VMEM((K,), jnp.int32),
          pltpu.VMEM((K,), jnp.float32),
          pltpu.VMEM_SHARED((nr,), jnp.float32),
          pltpu.SemaphoreType.DMA,
      ],
  )
  def prep(src_h, dst_h, typ_h, zeros_h, g_h, dstp_h, w_h,
           big_v, key_v, cnt_v, w_v, ones_v, zi_v, zf_v, hist_sh, sem):
    sid = lax.axis_index("s")
    cid = lax.axis_index("c")
    wid = sid * NC + cid
    pb = per_tile_b

    pltpu.sync_copy(zeros_h, cnt_v.at[pl.ds(0, stripe)])
    pltpu.sync_copy(cnt_v.at[pl.ds(0, stripe)],
                    hist_sh.at[pl.ds(sid * stripe, stripe)])
    for j in range(K // LANES):
      sl = pl.ds(j * LANES, LANES)
      ones_v[sl] = jnp.full((LANES,), 1.0, jnp.float32)
      zi_v[sl] = jnp.zeros((LANES,), jnp.int32)
      zf_v[sl] = jnp.zeros((LANES,), jnp.float32)

    base_a = sid * per_tile_a
    pltpu.sync_copy(dst_h.at[pl.ds(base_a, per_tile_a)],
                    big_v.at[pl.ds(0, per_tile_a)])
    pltpu.sync_copy(typ_h.at[pl.ds(base_a, per_tile_a)],
                    big_v.at[pl.ds(per_tile_a, per_tile_a)])
    plsc.subcore_barrier()

    def body_a(b, carry):
      for j in range(K // LANES):
        o = b * K + j * LANES
        d16 = big_v[pl.ds(o, LANES)]
        t16 = big_v[pl.ds(per_tile_a + o, LANES)]
        key_v[b, pl.ds(j * LANES, LANES)] = d16 * r + t16
      pltpu.async_copy(ones_v, hist_sh.at[key_v.at[b]], sem, add=True)
      return carry

    lax.fori_loop(0, nb_a, body_a, 0)
    pltpu.make_async_copy(dst_h.at[pl.ds(base_a, per_tile_a)],
                          big_v.at[pl.ds(0, per_tile_a)], sem).wait()
    plsc.subcore_barrier()

    base_b = wid * pb
    pltpu.sync_copy(src_h.at[pl.ds(base_b, pb)], big_v.at[pl.ds(0, pb)])
    pltpu.sync_copy(dst_h.at[pl.ds(base_b, pb)], big_v.at[pl.ds(pb, pb)])
    pltpu.sync_copy(typ_h.at[pl.ds(base_b, pb)], big_v.at[pl.ds(2 * pb, pb)])

    def body_b(b, carry):
      for j in range(K // LANES):
        o = b * K + j * LANES
        s16 = big_v[pl.ds(o, LANES)]
        d16 = big_v[pl.ds(pb + o, LANES)]
        t16 = big_v[pl.ds(2 * pb + o, LANES)]
        big_v[pl.ds(3 * pb + o, LANES)] = s16 * r + t16
        key_v[b, pl.ds(j * LANES, LANES)] = d16 * r + t16
      pltpu.async_copy(hist_sh.at[key_v.at[b]], cnt_v.at[pl.ds(b * K, K)], sem)
      return carry

    lax.fori_loop(0, nb_b, body_b, 0)
    del body_b
    pltpu.make_async_copy(src_h.at[pl.ds(base_b, pb)],
                          cnt_v, sem).wait()

    def body_w(i, carry):
      sl = pl.ds(i * LANES, LANES)
      w_v[sl] = 1.0 / jnp.maximum(cnt_v[sl], 1.0)
      return carry

    lax.fori_loop(0, lanes_b, body_w, 0)
    ob = wid * pt
    pltpu.sync_copy(big_v.at[pl.ds(3 * pb, pb)], g_h.at[pl.ds(ob, pb)])
    pltpu.sync_copy(big_v.at[pl.ds(pb, pb)], dstp_h.at[pl.ds(ob, pb)])
    pltpu.sync_copy(w_v, w_h.at[pl.ds(ob, pb)])
    for q in range(padw // K):
      pltpu.sync_copy(zi_v, g_h.at[pl.ds(ob + pb + q * K, K)])
      pltpu.sync_copy(zi_v, dstp_h.at[pl.ds(ob + pb + q * K, K)])
      pltpu.sync_copy(zf_v, w_h.at[pl.ds(ob + pb + q * K, K)])

  return prep(src, dst, typ, zeros_hist)


NBUF = 2
SCH = 18


@functools.partial(jax.jit, static_argnames=("n", "d"))
def _sc_scatter(table, g4, dst4, w4, zeros_acc, *, n, d):
  pt = g4.shape[0] // NW
  nbt = pt // K
  nsc = nbt // SCH
  ck = SCH * K
  total_chunks = n // K
  NZC = -(-total_chunks // NS)
  NZC_LAST = total_chunks - (NS - 1) * NZC

  @functools.partial(
      pl.kernel,
      out_type=jax.ShapeDtypeStruct((NC, n, d), jnp.float32),
      mesh=_mesh(),
      scratch_types=[
          pltpu.VMEM((NBUF * ck,), jnp.int32),
          pltpu.VMEM((NBUF * ck,), jnp.int32),
          pltpu.VMEM((NBUF * ck,), jnp.float32),
          pltpu.VMEM((K,), jnp.int32),
          pltpu.VMEM((K,), jnp.int32),
          pltpu.VMEM((NBUF, K, d // 2), jnp.int32),
          pltpu.VMEM((NBUF, K, d), jnp.float32),
          pltpu.VMEM_SHARED((n, d), jnp.float32),
          pltpu.SemaphoreType.DMA((NBUF,)),
          pltpu.SemaphoreType.DMA((NBUF,)),
          pltpu.SemaphoreType.DMA((NBUF,)),
      ],
      compiler_params=pltpu.CompilerParams(needs_layout_passes=False,
                                           use_tc_tiling_on_sc=False),
  )
  def scat(t_h, g_h, dst_h, w_h, zeros_h, out_h,
           g_v, dst_v, w_v, di0_v, di1_v, rows_v, rowf_v, acc_sh,
           gsem, ssem, msem):
    di_v = (di0_v, di1_v)
    sid = lax.axis_index("s")
    cid = lax.axis_index("c")
    wid = sid * NC + cid
    hbase = wid * pt

    pltpu.async_copy(g_h.at[pl.ds(hbase, ck)], g_v.at[pl.ds(0, ck)],
                     msem.at[0])
    pltpu.async_copy(dst_h.at[pl.ds(hbase, ck)], dst_v.at[pl.ds(0, ck)],
                     msem.at[0])
    pltpu.async_copy(w_h.at[pl.ds(hbase, ck)], w_v.at[pl.ds(0, ck)],
                     msem.at[0])

    pltpu.sync_copy(zeros_h, rowf_v.at[0])

    def zinit(c, carry):
      pltpu.sync_copy(rowf_v.at[0],
                      acc_sh.at[pl.ds(sid * (K * NZC) + c * K, K)])
      return carry

    nz = jnp.where(sid == NS - 1, NZC_LAST, NZC)
    lax.fori_loop(0, nz, zinit, 0)
    plsc.subcore_barrier()

    for s in range(nsc):
      m = s % NBUF
      if s > 0:
        pltpu.make_async_copy(rowf_v.at[1], acc_sh.at[di_v[1]],
                              ssem.at[1]).wait()
      if s + 1 < nsc:
        mn = (s + 1) % NBUF
        hoff = hbase + (s + 1) * ck
        pltpu.async_copy(g_h.at[pl.ds(hoff, ck)],
                         g_v.at[pl.ds(mn * ck, ck)], msem.at[mn])
        pltpu.async_copy(dst_h.at[pl.ds(hoff, ck)],
                         dst_v.at[pl.ds(mn * ck, ck)], msem.at[mn])
        pltpu.async_copy(w_h.at[pl.ds(hoff, ck)],
                         w_v.at[pl.ds(mn * ck, ck)], msem.at[mn])
      hcur = hbase + s * ck
      pltpu.make_async_copy(g_h.at[pl.ds(hcur, ck)],
                            g_v.at[pl.ds(m * ck, ck)], msem.at[m]).wait()
      pltpu.make_async_copy(dst_h.at[pl.ds(hcur, ck)],
                            dst_v.at[pl.ds(m * ck, ck)], msem.at[m]).wait()
      pltpu.make_async_copy(w_h.at[pl.ds(hcur, ck)],
                            w_v.at[pl.ds(m * ck, ck)], msem.at[m]).wait()
      pltpu.async_copy(t_h.at[g_v.at[pl.ds(m * ck, K)]], rows_v.at[0],
                       gsem.at[0])

      def inner(gi, carry, m=m):
        for j in range(NBUF):
          i = gi * NBUF + j
          jn = (j + 1) % NBUF

          for grp in range(K // LANES):
            di_v[j][pl.ds(grp * LANES, LANES)] = (
                dst_v[pl.ds(m * ck + i * K + grp * LANES, LANES)])

          @pl.when(i >= 1)
          def _():
            pltpu.make_async_copy(rowf_v.at[jn], acc_sh.at[di_v[jn]],
                                  ssem.at[jn]).wait()

          @pl.when(i + 1 < SCH)
          def _():
            pltpu.async_copy(t_h.at[g_v.at[pl.ds(m * ck + (i + 1) * K, K)]],
                             rows_v.at[jn], gsem.at[jn])

          pltpu.make_async_copy(t_h.at[g_v.at[pl.ds(m * ck + i * K, K)]],
                                rows_v.at[j], gsem.at[j]).wait()

          def scale(grp, c2):
            base_i = grp * LANES
            wv = w_v[pl.ds(m * ck + i * K + base_i, LANES)]
            for ii in range(LANES):
              ws = wv[ii]
              row = base_i + ii
              for c in range(d // (2 * LANES)):
                pair = rows_v[j, row, pl.ds(c * LANES, LANES)]
                ab = plsc.bitcast(pair, jnp.bfloat16)
                lo, hi = plsc.unpack(ab, format=plsc.PackFormat.INTERLEAVED)
                rowf_v[j, row, pl.ds(c * 2 * LANES, LANES)] = lo * ws
                rowf_v[j, row, pl.ds(c * 2 * LANES + LANES, LANES)] = hi * ws
            return c2

          lax.fori_loop(0, K // LANES, scale, 0)
          pltpu.async_copy(rowf_v.at[j], acc_sh.at[di_v[j]],
                           ssem.at[j], add=True)
        return carry

      lax.fori_loop(0, SCH // NBUF, inner, 0)

    pltpu.make_async_copy(rowf_v.at[1], acc_sh.at[di_v[1]],
                          ssem.at[1]).wait()
    plsc.subcore_barrier()

    def flush(c, carry):
      row0 = sid * (K * NZC) + c * K
      pltpu.sync_copy(acc_sh.at[pl.ds(row0, K)], rowf_v.at[0])
      pltpu.sync_copy(rowf_v.at[0], out_h.at[cid, pl.ds(row0, K)])
      return carry

    lax.fori_loop(0, nz, flush, 0)

  return scat(table, g4, dst4, w4, zeros_acc)


_BN = 1000


def _mm_out(h, rel_ref, root_ref, orel_ref, oroot_ref):
  r, d = rel_ref.shape[0], rel_ref.shape[1]
  for rr in range(r):
    orel_ref[:, pl.ds(rr * d, d)] = jnp.dot(
        h, rel_ref[rr], preferred_element_type=jnp.float32
    ).astype(jnp.bfloat16)
  oroot_ref[...] = jnp.dot(h, root_ref[...],
                           preferred_element_type=jnp.float32)


def _tc_first_body(x_ref, w_ref, b_ref, rel_ref, root_ref,
                   orel_ref, oroot_ref):
  h = jnp.dot(x_ref[...], w_ref[...], preferred_element_type=jnp.float32)
  h = jnp.maximum(h + b_ref[...], 0.0)
  _mm_out(h, rel_ref, root_ref, orel_ref, oroot_ref)


def _tc_first(x, w_in, b_in, rel, root):
  n, d = x.shape
  r = rel.shape[0]
  return pl.pallas_call(
      _tc_first_body,
      grid=(n // _BN,),
      in_specs=[
          pl.BlockSpec((_BN, d), lambda i: (i, 0)),
          pl.BlockSpec((d, d), lambda i: (0, 0)),
          pl.BlockSpec((1, d), lambda i: (0, 0)),
          pl.BlockSpec((r, d, d), lambda i: (0, 0, 0)),
          pl.BlockSpec((d, d), lambda i: (0, 0)),
      ],
      out_specs=(pl.BlockSpec((_BN, r * d), lambda i: (i, 0)),
                 pl.BlockSpec((_BN, d), lambda i: (i, 0))),
      out_shape=(jax.ShapeDtypeStruct((n, r * d), jnp.bfloat16),
                 jax.ShapeDtypeStruct((n, d), jnp.float32)),
  )(x, w_in, b_in, rel, root)


def _tc_mid_body(p_ref, root_ref, b_ref, rel_ref, rootw_ref,
                 orel_ref, oroot_ref):
  p = p_ref[...]
  h = jnp.maximum(p[0] + p[1] + root_ref[...] + b_ref[...], 0.0)
  _mm_out(h, rel_ref, rootw_ref, orel_ref, oroot_ref)


def _tc_mid(p, t_root, b, rel, root):
  n, d = p.shape[1], p.shape[2]
  r = rel.shape[0]
  return pl.pallas_call(
      _tc_mid_body,
      grid=(n // _BN,),
      in_specs=[
          pl.BlockSpec((2, _BN, d), lambda i: (0, i, 0)),
          pl.BlockSpec((_BN, d), lambda i: (i, 0)),
          pl.BlockSpec((1, d), lambda i: (0, 0)),
          pl.BlockSpec((r, d, d), lambda i: (0, 0, 0)),
          pl.BlockSpec((d, d), lambda i: (0, 0)),
      ],
      out_specs=(pl.BlockSpec((_BN, r * d), lambda i: (i, 0)),
                 pl.BlockSpec((_BN, d), lambda i: (i, 0))),
      out_shape=(jax.ShapeDtypeStruct((n, r * d), jnp.bfloat16),
                 jax.ShapeDtypeStruct((n, d), jnp.float32)),
  )(p, t_root, b, rel, root)


def _tc_last_body(p_ref, root_ref, b_ref, o_ref):
  p = p_ref[...]
  o_ref[...] = jnp.maximum(p[0] + p[1] + root_ref[...] + b_ref[...], 0.0)


def _tc_last(p, t_root, b):
  n, d = p.shape[1], p.shape[2]
  return pl.pallas_call(
      _tc_last_body,
      grid=(n // _BN,),
      in_specs=[
          pl.BlockSpec((2, _BN, d), lambda i: (0, i, 0)),
          pl.BlockSpec((_BN, d), lambda i: (i, 0)),
          pl.BlockSpec((1, d), lambda i: (0, 0)),
      ],
      out_specs=pl.BlockSpec((_BN, d), lambda i: (i, 0)),
      out_shape=jax.ShapeDtypeStruct((n, d), jnp.float32),
  )(p, t_root, b)


def kernel(x, edge_index, edge_attr, W_in, b_in, rel_W, root_W, root_b):
  n, d = x.shape
  e = edge_index.shape[1]
  nl, r = rel_W.shape[0], rel_W.shape[1]

  src = edge_index[0]
  dst = edge_index[1]
  typ = edge_attr[:, 1].astype(jnp.int32)
  edge_distance = edge_attr[:, 0].astype(jnp.float32)

  zeros_hist = jnp.zeros((n * r // NS,), jnp.float32)
  zeros_acc = jnp.zeros((K, d), jnp.float32)

  nb = e // NW // K
  nbt = -(-nb // SCH) * SCH
  pt = nbt * K

  g4, dst4, w4 = _sc_prep(src, dst, typ, zeros_hist, n=n, r=r, pt=pt)

  q = np.arange(d).reshape(d // 32, 2, 16).transpose(0, 2, 1).reshape(-1)
  rel_q = rel_W[:, :, :, q]

  t_rel, t_root = _tc_first(x, W_in, b_in.reshape(1, d),
                            rel_q[0], root_W[0])
  h = None
  for l in range(nl):
    t32 = lax.bitcast_convert_type(
        t_rel.reshape(n * r, d // 2, 2), jnp.int32)
    p = _sc_scatter(t32, g4, dst4, w4, zeros_acc, n=n, d=d)
    if l < nl - 1:
      t_rel, t_root = _tc_mid(p, t_root, root_b[l].reshape(1, d),
                              rel_q[l + 1], root_W[l + 1])
    else:
      h = _tc_last(p, t_root, root_b[l].reshape(1, d))
  return (h, edge_distance)
